# baseline (speedup 1.0000x reference)
import functools
import math
INTERPRET = False

import jax
import jax.numpy as jnp
from jax import lax
from jax.experimental import pallas as pl
from jax.experimental.pallas import tpu as pltpu
from jax.experimental.pallas import tpu_sc as plsc

D = 128
B = 16
S = 4096
NC = 2
NS = 16
L = 16
SHALF = S // NC
CH = 256
NCHUNK = SHALF // CH
NJ = D // L


def _sc_body(in0_hbm, in1_hbm, v_hbm, parts_hbm, ml_hbm,
             buf0, buf1, v_ref, scores_ref, w_ref, out_ref, ml_ref):
    b = lax.axis_index("s")
    h = lax.axis_index("c")
    wid = b * NC + h

    pltpu.sync_copy(v_hbm, v_ref)
    vs = [v_ref[pl.ds(j * L, L)] for j in range(NJ)]
    lane = lax.iota(jnp.int32, L)

    zeros = jnp.zeros((L,), jnp.float32)
    accs0 = tuple(zeros for _ in range(2 * NJ))
    m0 = jnp.float32(-1e30)
    l0 = zeros
    row0 = h * SHALF

    @pl.loop(0, NCHUNK, init_carry=(m0, l0, accs0))
    def chunk_loop(c, carry):
        m, l16, accs = carry
        start = row0 + c * CH
        pltpu.sync_copy(in1_hbm.at[b, pl.ds(start, CH)], buf1)

        @pl.loop(0, CH // L)
        def score_groups(g):
            def row_score(r, sc16):
                s = g * L + r
                acc = vs[0] * buf1[s, pl.ds(0, L)]
                for j in range(1, NJ):
                    acc = acc + vs[j] * buf1[s, pl.ds(j * L, L)]
                sc = jnp.sum(acc)
                return jnp.where(lane == r, sc, sc16)
            scores_ref[pl.ds(g * L, L)] = lax.fori_loop(
                0, L, row_score, zeros)

        def group_max(g, m16):
            return jnp.maximum(m16, scores_ref[pl.ds(g * L, L)])
        m16 = lax.fori_loop(0, CH // L, group_max,
                            jnp.full((L,), -1e30, jnp.float32))
        mc = jnp.max(m16)
        m_new = jnp.maximum(m, mc)
        rs16 = jnp.exp(jnp.full((L,), m - m_new, jnp.float32))
        l16 = l16 * rs16
        accs = tuple(a * rs16 for a in accs)
        m_full = jnp.full((L,), m_new, jnp.float32)

        def group_w(g, l16):
            w16 = jnp.exp(scores_ref[pl.ds(g * L, L)] - m_full)
            w_ref[pl.ds(g * L, L)] = w16
            return l16 + w16
        l16 = lax.fori_loop(0, CH // L, group_w, l16)

        pltpu.sync_copy(in0_hbm.at[b, pl.ds(start, CH)], buf0)

        def row_acc(s, accs):
            wspl = plsc.load_gather(w_ref, [jnp.full((L,), s, jnp.int32)])
            new = []
            for j in range(NJ):
                new.append(accs[j] + wspl * buf0[s, pl.ds(j * L, L)])
            for j in range(NJ):
                new.append(accs[NJ + j] + wspl * buf1[s, pl.ds(j * L, L)])
            return tuple(new)
        accs = lax.fori_loop(0, CH, row_acc, accs)
        return m_new, l16, accs

    m, l16, accs = chunk_loop
    lsum = jnp.sum(l16)
    for j in range(2 * NJ):
        out_ref[pl.ds(j * L, L)] = accs[j]
    ml_ref[...] = jnp.where(lane == 0, jnp.full((L,), m, jnp.float32),
                            jnp.where(lane == 1, jnp.full((L,), lsum, jnp.float32),
                                      zeros))
    pltpu.sync_copy(out_ref, parts_hbm.at[wid])
    pltpu.sync_copy(ml_ref, ml_hbm.at[wid])


@jax.jit
def kernel(inputs_0, inputs_1, W, b, fingerprint):
    scale = math.sqrt(float(D))
    v = (jnp.diagonal(W) * fingerprint / scale).astype(jnp.float32)

    mesh = plsc.VectorSubcoreMesh(
        core_axis_name="c", subcore_axis_name="s",
        num_cores=NC, num_subcores=NS)
    sc_call = pl.kernel(
        _sc_body,
        out_type=[
            jax.ShapeDtypeStruct((NC * NS, 2 * D), jnp.float32),
            jax.ShapeDtypeStruct((NC * NS, L), jnp.float32),
        ],
        mesh=mesh,
        interpret=INTERPRET,
        compiler_params=pltpu.CompilerParams(needs_layout_passes=False),
        scratch_types=[
            pltpu.VMEM((CH, D), jnp.float32),
            pltpu.VMEM((CH, D), jnp.float32),
            pltpu.VMEM((D,), jnp.float32),
            pltpu.VMEM((CH,), jnp.float32),
            pltpu.VMEM((CH,), jnp.float32),
            pltpu.VMEM((2 * D,), jnp.float32),
            pltpu.VMEM((L,), jnp.float32),
        ],
    )
    parts, ml = sc_call(inputs_0, inputs_1, v)

    parts = parts.reshape(B, NC, 2 * D)
    m = ml[:, 0].reshape(B, NC)
    l = ml[:, 1].reshape(B, NC)
    mb = jnp.max(m, axis=1)
    coef = jnp.exp(m - mb[:, None])
    denom = jnp.sum(l * coef, axis=1)
    out = jnp.sum(parts * coef[:, :, None], axis=1) / denom[:, None]
    return out


# speedup vs baseline: 1.0775x; 1.0775x over previous
import math

import jax
import jax.numpy as jnp
from jax import lax
from jax.experimental import pallas as pl
from jax.experimental.pallas import tpu as pltpu
from jax.experimental.pallas import tpu_sc as plsc

D = 128
B = 16
S = 4096
NC = 2
NS = 16
L = 16
SHALF = S // 2
CH = 128
NCHUNK = SHALF // CH
NJ = D // L
PCOLS = 2 * D + L


def _sc_body(in0_hbm, in1_hbm, w_hbm, fp_hbm, out_hbm,
             buf0, buf1, wmat, fp_ref, stage, shared, part,
             sem00, sem01, sem10, sem11):
    s_idx = lax.axis_index("s")
    c_idx = lax.axis_index("c")
    b = c_idx * (NS // 2) + s_idx // 2
    h = s_idx % 2
    lane = lax.iota(jnp.int32, L)

    pltpu.sync_copy(w_hbm, wmat)
    pltpu.sync_copy(fp_hbm, fp_ref)
    inv_scale = 1.0 / math.sqrt(float(D))
    vs = []
    for j in range(NJ):
        diag_idx = lane + j * L
        dj = plsc.load_gather(wmat, [diag_idx, diag_idx])
        vs.append(dj * fp_ref[pl.ds(j * L, L)] * inv_scale)

    zeros = jnp.zeros((L,), jnp.float32)
    accs0 = tuple(zeros for _ in range(2 * NJ))
    m0 = jnp.float32(-1e30)
    l0 = zeros
    row0 = h * SHALF
    sems0 = (sem00, sem01)
    sems1 = (sem10, sem11)

    def issue(c, slot):
        start = row0 + c * CH
        pltpu.async_copy(in1_hbm.at[b, pl.ds(start, CH)], buf1.at[slot],
                         sems1[slot])
        pltpu.async_copy(in0_hbm.at[b, pl.ds(start, CH)], buf0.at[slot],
                         sems0[slot])

    issue(0, 0)

    @pl.loop(0, NCHUNK // 2, init_carry=(m0, l0, accs0))
    def chunk_loop(i, carry):
        m, l16, accs = carry
        for slot in range(2):
            c = 2 * i + slot

            @pl.when(c + 1 < NCHUNK)
            def _():
                issue(c + 1, 1 - slot)

            b1 = buf1.at[slot]
            b0 = buf0.at[slot]
            pltpu.make_async_copy(in1_hbm.at[b, pl.ds(row0, CH)],
                                  b1, sems1[slot]).wait()
            pltpu.make_async_copy(in0_hbm.at[b, pl.ds(row0, CH)],
                                  b0, sems0[slot]).wait()

            def row_step(r, carry):
                m, l16, accs = carry
                x1 = [b1[r, pl.ds(j * L, L)] for j in range(NJ)]
                p = x1[0] * vs[0]
                for j in range(1, NJ):
                    p = p + x1[j] * vs[j]
                sc = jnp.sum(p)
                d = sc - m
                m = jnp.maximum(m, sc)
                w16 = jnp.exp(jnp.full((L,), jnp.minimum(d, 0.0),
                                       jnp.float32))
                e16 = jnp.exp(jnp.full((L,), jnp.minimum(-d, 0.0),
                                       jnp.float32))
                l16 = l16 * e16 + w16
                new = []
                for j in range(NJ):
                    new.append(accs[j] * e16 + w16 * b0[r, pl.ds(j * L, L)])
                for j in range(NJ):
                    new.append(accs[NJ + j] * e16 + w16 * x1[j])
                return m, l16, tuple(new)

            m, l16, accs = lax.fori_loop(0, CH, row_step, (m, l16, accs),
                                         unroll=2)
        return m, l16, accs

    m, l16, accs = chunk_loop

    for j in range(2 * NJ):
        stage[pl.ds(j * L, L)] = accs[j]
    mf = jnp.full((L,), m, jnp.float32)
    stage[pl.ds(2 * D, L)] = jnp.where(lane == 0, mf,
                                       jnp.where(lane == 1, l16, zeros))
    pltpu.sync_copy(stage, shared.at[s_idx])
    plsc.subcore_barrier()

    @pl.when(h == 0)
    def _():
        pltpu.sync_copy(shared.at[s_idx + 1], part)
        ml_o = part[pl.ds(2 * D, L)]
        m_o = plsc.load_gather(part, [jnp.full((L,), 2 * D, jnp.int32)])
        l_o = plsc.load_gather(part, [jnp.full((L,), 2 * D + 1, jnp.int32)])
        m_s = jnp.full((L,), m, jnp.float32)
        mb = jnp.maximum(m_s, m_o)
        cs = jnp.exp(m_s - mb)
        co = jnp.exp(m_o - mb)
        denom = l16 * cs + l_o * co
        r = 1.0 / denom
        for j in range(2 * NJ):
            acc_o = part[pl.ds(j * L, L)]
            stage[pl.ds(j * L, L)] = (accs[j] * cs + acc_o * co) * r
        pltpu.sync_copy(stage.at[pl.ds(0, 2 * D)], out_hbm.at[b])


@jax.jit
def kernel(inputs_0, inputs_1, W, b, fingerprint):
    mesh = plsc.VectorSubcoreMesh(
        core_axis_name="c", subcore_axis_name="s",
        num_cores=NC, num_subcores=NS)
    sc_call = pl.kernel(
        _sc_body,
        out_type=jax.ShapeDtypeStruct((B, 2 * D), jnp.float32),
        mesh=mesh,
        compiler_params=pltpu.CompilerParams(needs_layout_passes=False),
        scratch_types=[
            pltpu.VMEM((2, CH, D), jnp.float32),
            pltpu.VMEM((2, CH, D), jnp.float32),
            pltpu.VMEM((D, D), jnp.float32),
            pltpu.VMEM((D,), jnp.float32),
            pltpu.VMEM((PCOLS,), jnp.float32),
            pltpu.VMEM_SHARED((NS, PCOLS), jnp.float32),
            pltpu.VMEM((PCOLS,), jnp.float32),
            pltpu.SemaphoreType.DMA,
            pltpu.SemaphoreType.DMA,
            pltpu.SemaphoreType.DMA,
            pltpu.SemaphoreType.DMA,
        ],
    )
    return sc_call(inputs_0, inputs_1, W, fingerprint)
